# baseline (speedup 1.0000x reference)
import functools

import jax
import jax.numpy as jnp
from jax import lax
from jax.experimental import pallas as pl
from jax.experimental.pallas import tpu as pltpu
from jax.experimental.pallas import tpu_sc as plsc

N = 10000
E = 160000
NP = 10240
EP = 163840


def _row_call(fn, out_shapes, row_args, const_args=(), block=2048):
    R = row_args[0].shape[0]
    while R % block:
        block //= 2
    grid = (R // block,)

    def row_spec(a):
        nd = a.ndim
        return pl.BlockSpec((block,) + a.shape[1:],
                            lambda i, nd=nd: (i,) + (0,) * (nd - 1))

    def const_spec(a):
        nd = a.ndim
        return pl.BlockSpec(a.shape, lambda i, nd=nd: (0,) * nd)

    in_specs = [row_spec(a) for a in row_args] + [const_spec(c) for c in const_args]
    out_specs = tuple(pl.BlockSpec((block,) + s[1:],
                                   lambda i, nd=len(s): (i,) + (0,) * (nd - 1))
                      for s in out_shapes)
    out_shape = tuple(jax.ShapeDtypeStruct(s, jnp.float32) for s in out_shapes)
    n_in = len(row_args) + len(const_args)

    def body(*refs):
        ins = [r[...] for r in refs[:n_in]]
        outs = fn(*ins)
        if not isinstance(outs, (tuple, list)):
            outs = (outs,)
        for oref, val in zip(refs[n_in:], outs):
            oref[...] = val

    res = pl.pallas_call(
        body, grid=grid, in_specs=in_specs, out_specs=out_specs,
        out_shape=out_shape)(*row_args, *const_args)
    return res


def _row_reduce_sum(fn, row_args, const_args=(), block=2048):
    R = row_args[0].shape[0]
    grid = (R // block,)

    def row_spec(a):
        nd = a.ndim
        return pl.BlockSpec((block,) + a.shape[1:],
                            lambda i, nd=nd: (i,) + (0,) * (nd - 1))

    def const_spec(a):
        nd = a.ndim
        return pl.BlockSpec(a.shape, lambda i, nd=nd: (0,) * nd)

    in_specs = [row_spec(a) for a in row_args] + [const_spec(c) for c in const_args]
    n_in = len(row_args) + len(const_args)

    def body(*refs):
        i = pl.program_id(0)
        out = refs[-1]
        @pl.when(i == 0)
        def _():
            out[...] = jnp.zeros_like(out)
        ins = [r[...] for r in refs[:n_in]]
        out[...] += jnp.sum(fn(*ins)).reshape(1, 1)

    return pl.pallas_call(
        body, grid=grid, in_specs=in_specs,
        out_specs=pl.BlockSpec((1, 1), lambda i: (0, 0)),
        out_shape=jax.ShapeDtypeStruct((1, 1), jnp.float32))(*row_args, *const_args)


def _qmul(a, b):
    aw, ax, ay, az = a[:, 0], a[:, 1], a[:, 2], a[:, 3]
    bw, bx, by, bz = b[:, 0], b[:, 1], b[:, 2], b[:, 3]
    return jnp.stack([
        aw * bw - ax * bx - ay * by - az * bz,
        aw * bx + ax * bw + ay * bz - az * by,
        aw * by - ax * bz + ay * bw + az * bx,
        aw * bz + ax * by - ay * bx + az * bw], axis=1)


def _qmul_inv_a(a, b):
    ai = jnp.stack([a[:, 0], -a[:, 1], -a[:, 2], -a[:, 3]], axis=1)
    return _qmul(ai, b)


def _l2n(v):
    return v / (jnp.sqrt(jnp.sum(v * v, axis=1, keepdims=True)) + 1e-8)


def _node_mm(x, W1, W2):
    def pad64(W):
        return jnp.pad(W, ((0, 0), (0, 64 - W.shape[1])))
    W12 = jnp.concatenate([pad64(W1), pad64(W2)], axis=1)
    return _row_call(
        lambda xb, w: jnp.dot(xb, w, preferred_element_type=jnp.float32),
        [(x.shape[0], 128)], [x], [W12])[0]


def _node_finish(aggp, cnts, F):
    NT = aggp.shape[0] // 2
    ccol = cnts.shape[1] - 16
    return _row_call(
        lambda a0, a1, c0, c1: jax.nn.relu(
            (a0 + a1)[:, :F] / jnp.maximum((c0 + c1)[:, ccol:ccol + 1], 1.0)),
        [(NT, F)], [aggp[:NT], aggp[NT:], cnts[:NT], cnts[NT:]])[0]


def _edge_p(ea_list, W3, b, mask=None, mask_mult=False, cnt_col=False):
    F = W3.shape[1]
    nea = len(ea_list)
    Fo = F + 16 if cnt_col else F

    def fn(*args):
        p = jnp.dot(jnp.concatenate(args[:nea], axis=1), args[-2],
                    preferred_element_type=jnp.float32) + args[-1]
        if mask_mult:
            p = p * args[nea]
        if cnt_col:
            z = jnp.zeros((p.shape[0], 15), jnp.float32)
            p = jnp.concatenate([p, args[nea], z], axis=1)
        return p
    rows = list(ea_list) + ([mask] if mask is not None else [])
    return _row_call(fn, [(ea_list[0].shape[0], Fo)], rows,
                     [W3, b.reshape(1, -1)])[0]


def _relu_rows(a, F):
    return _row_call(lambda v: jax.nn.relu(v[:, :F]), [(a.shape[0], F)], [a])[0]


CH = 128
NW = 32
NCH = EP // CH // NW


def _conv_scatter(tab, p, row2d, col2d, NT, F):
    Fp = p.shape[1]
    has_cnt = Fp == F + 16
    AW = Fp
    rows_pt = NT // 16
    assert rows_pt % 32 == 0 and F % 16 == 0
    mesh = plsc.VectorSubcoreMesh(core_axis_name="c", subcore_axis_name="s")
    out_type = (jax.ShapeDtypeStruct((EP, F), jnp.float32),
                jax.ShapeDtypeStruct((2 * NT, AW), jnp.float32))
    scratch = [
        pltpu.VMEM((NCH, CH), jnp.int32),
        pltpu.VMEM((NCH, CH), jnp.int32),
        pltpu.VMEM((CH, 128), jnp.float32),
        pltpu.VMEM((CH, 128), jnp.float32),
        pltpu.VMEM((CH, Fp), jnp.float32),
        pltpu.VMEM((CH, AW), jnp.float32),
        pltpu.VMEM((32, AW), jnp.float32),
        pltpu.VMEM_SHARED((NT, AW), jnp.float32),
        pltpu.SemaphoreType.DMA,
        pltpu.SemaphoreType.DMA,
    ]
    if has_cnt:
        scratch.insert(6, pltpu.VMEM((CH, F), jnp.float32))

    def body(*refs):
        if has_cnt:
            (tab_h, p_h, r_h, c_h, e_h, agg_h,
             rbuf, cbuf, gr, gc, pb, sb, eb, zb, agg_sh, sem1, sem2) = refs
        else:
            (tab_h, p_h, r_h, c_h, e_h, agg_h,
             rbuf, cbuf, gr, gc, pb, sb, zb, agg_sh, sem1, sem2) = refs
            eb = sb
        cid = lax.axis_index("c")
        sid = lax.axis_index("s")
        w = sid * 2 + cid
        ebase = w * NCH

        def zrow32(r, _):
            for q in range(AW // 16):
                zb[r, pl.ds(q * 16, 16)] = jnp.zeros((16,), jnp.float32)
            return 0
        lax.fori_loop(0, 32, zrow32, 0)
        for k in range(rows_pt // 32):
            pltpu.sync_copy(zb, agg_sh.at[pl.ds(sid * rows_pt + k * 32, 32)])
        plsc.subcore_barrier()

        pltpu.sync_copy(r_h.at[pl.ds(ebase, NCH)], rbuf)
        pltpu.sync_copy(c_h.at[pl.ds(ebase, NCH)], cbuf)

        def chunk(j, _):
            base = (ebase + j) * CH
            cp1 = pltpu.async_copy(tab_h.at[rbuf.at[j]], gr, sem1)
            cp2 = pltpu.async_copy(tab_h.at[cbuf.at[j]], gc, sem2)
            pltpu.sync_copy(p_h.at[pl.ds(base, CH)], pb)
            cp1.wait()
            cp2.wait()

            def crow(r, _):
                for q in range(F // 16):
                    s = pl.ds(q * 16, 16)
                    v = gr[r, s] + gc[r, pl.ds(64 + q * 16, 16)] + pb[r, s]
                    sb[r, s] = v
                    if has_cnt:
                        eb[r, s] = v
                if has_cnt:
                    sb[r, pl.ds(F, 16)] = pb[r, pl.ds(F, 16)]
                return 0
            lax.fori_loop(0, CH, crow, 0)

            pltpu.sync_copy(eb, e_h.at[pl.ds(base, CH)])
            pltpu.sync_copy(sb, agg_sh.at[cbuf.at[j]], add=True)
            return 0
        lax.fori_loop(0, NCH, chunk, 0)
        plsc.subcore_barrier()

        for k in range(rows_pt // 32):
            r0 = sid * rows_pt + k * 32
            pltpu.sync_copy(agg_sh.at[pl.ds(r0, 32)], agg_h.at[pl.ds(cid * NT + r0, 32)])

    kfn = pl.kernel(body, out_type=out_type, mesh=mesh, scratch_types=scratch,
                    compiler_params=pltpu.CompilerParams(use_tc_tiling_on_sc=False))
    return kfn(tab, p, row2d, col2d)


def _gather_rows(tab, idx):
    return tab[idx]


def _scatter_rows_add(base, idx, delta):
    return base.at[idx].add(delta)


def _gat_scores(Wg, a_s, a_d, x, rowi, coli, m, n_out):
    h = (x @ Wg)[:, 0]
    e = jax.nn.leaky_relu(a_s * h[rowi] + a_d * h[coli], 0.2)
    e = jnp.where(m > 0, e, -1e9)
    mx = jnp.full((n_out,), -1e9, jnp.float32).at[coli].max(e)
    ex = jnp.exp(e - mx[coli]) * m
    den = jnp.zeros((n_out,), jnp.float32).at[coli].add(ex)
    alpha = ex / jnp.maximum(den[coli], 1e-9)
    gat = jnp.zeros((n_out,), jnp.float32).at[coli].add(alpha * h[rowi])
    return jnp.tanh(gat)


def _pool_remap(perm, k, rowi, coli, m, n_nodes):
    kept = jnp.zeros((n_nodes,), bool).at[perm].set(True)
    nid = jnp.zeros((n_nodes,), jnp.int32).at[perm].set(jnp.arange(k, dtype=jnp.int32))
    valid = kept[jnp.minimum(rowi, n_nodes - 1)] & kept[jnp.minimum(coli, n_nodes - 1)] & (m > 0)
    row2 = jnp.where(valid, nid[jnp.minimum(rowi, n_nodes - 1)], k)
    col2 = jnp.where(valid, nid[jnp.minimum(coli, n_nodes - 1)], k)
    fm = valid.astype(jnp.float32)
    return row2, col2, fm


def kernel(x_org, edge_index, edge_attr, gt_q, beta, params):
    p_ = params
    rowi = jnp.concatenate([edge_index[0], jnp.full((EP - E,), N, jnp.int32)])
    coli = jnp.concatenate([edge_index[1], jnp.full((EP - E,), N, jnp.int32)])
    ones_m = jnp.concatenate([jnp.ones((E,), jnp.float32), jnp.zeros((EP - E,), jnp.float32)])
    row2d, col2d = rowi.reshape(-1, CH), coli.reshape(-1, CH)
    ones2d = ones_m.reshape(-1, CH)
    ea8 = jnp.pad(edge_attr, ((0, EP - E), (0, 0)))
    x_org_p = jnp.pad(x_org, ((0, NP - N), (0, 0)))
    gt_q_p = jnp.pad(gt_q, ((0, NP - N), (0, 0)))
    k1, k2 = N // 2, N // 4
    K1P, K2P = 5120, 2560

    x0r = _gather_rows(x_org_p, rowi)
    x0c = _gather_rows(x_org_p, coli)
    eam = _row_call(
        lambda a, bq, e4: _l2n(_qmul(e4, _qmul_inv_a(a, bq))),
        [(EP, 4)], [x0r, x0c, ea8[:, :4]])[0]

    W, b = p_['c1W'], p_['c1b']
    ones_c = ones_m.reshape(-1, 1)
    tab = _node_mm(x_org_p, W[0:4], W[4:8])
    p1 = _edge_p([eam], W[8:12], b, mask=ones_c, cnt_col=True)
    e1p, agg1p = _conv_scatter(tab, p1, row2d, col2d, NP, 32)
    x1 = _node_finish(agg1p, agg1p, 32)
    e1r = _relu_rows(e1p, 32)

    W, b = p_['c2W'], p_['c2b']
    tab = _node_mm(x1, W[0:32], W[32:64])
    p2 = _edge_p([eam, e1r], W[64:100], b)
    e2p, aggp = _conv_scatter(tab, p2, row2d, col2d, NP, 32)
    x2 = _node_finish(aggp, agg1p, 32)
    e2r = _relu_rows(e2p, 32)

    W, b = p_['c3W'], p_['c3b']
    xc = jnp.concatenate([x2, x1], axis=1)
    tab = _node_mm(xc, W[0:64], W[64:128])
    p3 = _edge_p([e2r, e1r], W[128:192], b)
    e3p, aggp = _conv_scatter(tab, p3, row2d, col2d, NP, 32)
    x3 = _node_finish(aggp, agg1p, 32)
    e3r = _relu_rows(e3p, 32)

    W, b = p_['spW'], p_['spb']
    tab = _node_mm(x3, W[0:32], W[32:64])
    psp = _edge_p([e3r], W[64:96], b)
    esp_p, aggp = _conv_scatter(tab, psp, row2d, col2d, NP, 64)
    xs1 = _node_finish(aggp, agg1p, 64)
    es1r = _relu_rows(esp_p, 64)

    score1 = _gat_scores(p_['g1Wg'], p_['g1as'], p_['g1ad'], xs1, rowi, coli, ones_m, NP)
    vals1, perm1 = lax.top_k(score1[:N], k1)
    row2, col2, fm1 = _pool_remap(perm1, k1, rowi, coli, ones_m, N)
    xs1g = _gather_rows(xs1, perm1)
    xs1p = jnp.pad(xs1g * vals1[:, None], ((0, K1P - k1), (0, 0)))

    W, b = p_['s1W'], p_['s1b']
    fm1c = fm1.reshape(-1, 1)
    row2d2, col2d2 = row2.reshape(-1, CH), col2.reshape(-1, CH)
    tab = _node_mm(xs1p, W[0:64], W[64:128])
    ps1 = _edge_p([es1r], W[128:192], b, mask=fm1c, mask_mult=True, cnt_col=True)
    es2_p, aggs1p = _conv_scatter(tab, ps1, row2d2, col2d2, K1P, 64)
    xs2 = _node_finish(aggs1p, aggs1p, 64)
    es2r = _relu_rows(es2_p, 64)

    score2 = _gat_scores(p_['g2Wg'], p_['g2as'], p_['g2ad'], xs2, row2, col2, fm1, K1P)
    vals2, perm2 = lax.top_k(score2[:k1], k2)
    row3, col3, fm2 = _pool_remap(perm2, k2, row2, col2, fm1, k1)
    xs2g = _gather_rows(xs2, perm2)
    xssp = jnp.pad(xs2g * vals2[:, None], ((0, K2P - k2), (0, 0)))

    W, b = p_['ssW'], p_['ssb']
    fm2c = fm2.reshape(-1, 1)
    row2d3, col2d3 = row3.reshape(-1, CH), col3.reshape(-1, CH)
    tab = _node_mm(xssp, W[0:64], W[64:128])
    pss1 = _edge_p([es2r], W[128:192], b, mask=fm2c, mask_mult=True, cnt_col=True)
    ess1_p, aggss1p = _conv_scatter(tab, pss1, row2d3, col2d3, K2P, 64)
    xss1 = _node_finish(aggss1p, aggss1p, 64)
    ess1r = _relu_rows(ess1_p, 64)

    tab = _node_mm(xss1, W[0:64], W[64:128])
    pss2 = _edge_p([ess1r], W[128:192], b, mask=fm2c, mask_mult=True)
    ess2_p, aggp = _conv_scatter(tab, pss2, row2d3, col2d3, K2P, 64)
    xss2 = _node_finish(aggp, aggss1p, 64)

    xs2m = _scatter_rows_add(xs2, perm2, xss2[:k2])

    W, b = p_['s2W'], p_['s2b']
    tab = _node_mm(xs2m, W[0:64], W[64:128])
    ps2 = _edge_p([es2r], W[128:192], b, mask=fm1c, mask_mult=True)
    es3_p, aggp = _conv_scatter(tab, ps2, row2d2, col2d2, K1P, 32)
    xs3 = _node_finish(aggp, aggs1p, 32)

    x3b = _scatter_rows_add(x3, perm1, xs3[:k1])

    W, b = p_['c4W'], p_['c4b']
    xc4 = jnp.concatenate([x3b, x2], axis=1)
    tab = _node_mm(xc4, W[0:64], W[64:128])
    p4 = _edge_p([e3r, e2r], W[128:192], b)
    e4p, aggp = _conv_scatter(tab, p4, row2d, col2d, NP, 32)
    x4 = _node_finish(aggp, agg1p, 32)
    e4r = _relu_rows(e4p, 32)

    xc5 = jnp.concatenate([x4, x3b], axis=1)
    tab = _node_mm(xc5, W[0:64], W[64:128])
    p5 = _edge_p([e4r, e3r], W[128:192], b)
    e5p, aggp = _conv_scatter(tab, p5, row2d, col2d, NP, 32)
    x5 = _node_finish(aggp, agg1p, 32)
    e5r = _relu_rows(e5p, 32)

    xq = _row_call(
        lambda xv, xo, w, bb: _l2n(_qmul(jnp.dot(xv, w, preferred_element_type=jnp.float32) + bb, xo)),
        [(NP, 4)], [x5, x_org_p], [p_['l1W'], p_['l1b'].reshape(1, -1)])[0]

    gqr = _gather_rows(gt_q_p, rowi)
    gqc = _gather_rows(gt_q_p, coli)
    xr = _gather_rows(xq, rowi)
    xc_ = _gather_rows(xq, coli)

    def loss_fn(gr, gc, ar, ac, mv, bt):
        qa = _qmul_inv_a(gr, gc)
        qb = _qmul_inv_a(ar, ac)
        lv = _l2n(_qmul_inv_a(qa, qb))
        d = jnp.abs(jnp.stack([lv[:, 0] - 1.0, lv[:, 1], lv[:, 2], lv[:, 3]], axis=1))
        bb = jnp.maximum(bt[0, 0], 1e-6)
        contrib = jnp.where(d < bb, 0.5 * d * d / bb, d - 0.5 * bb)
        return contrib * mv

    loss_sum = _row_reduce_sum(loss_fn, [gqr, gqc, xr, xc_, ones_m.reshape(-1, 1)],
                               [beta.reshape(1, 1)])
    loss1 = loss_sum[0, 0] / (E * 4.0)

    return (xq[:N], loss1, beta,
            (x1[:N], x2[:N], x3b[:N], x4[:N], x5[:N]),
            (e1r[:E], e2r[:E], e3r[:E], e4r[:E], e5r[:E]))


# speedup vs baseline: 1.0617x; 1.0617x over previous
import functools

import jax
import jax.numpy as jnp
from jax import lax
from jax.experimental import pallas as pl
from jax.experimental.pallas import tpu as pltpu
from jax.experimental.pallas import tpu_sc as plsc

N = 10000
E = 160000
NP = 10240
EP = 163840


def _row_call(fn, out_shapes, row_args, const_args=(), block=2048):
    R = row_args[0].shape[0]
    while R % block:
        block //= 2
    grid = (R // block,)

    def row_spec(a):
        nd = a.ndim
        return pl.BlockSpec((block,) + a.shape[1:],
                            lambda i, nd=nd: (i,) + (0,) * (nd - 1))

    def const_spec(a):
        nd = a.ndim
        return pl.BlockSpec(a.shape, lambda i, nd=nd: (0,) * nd)

    in_specs = [row_spec(a) for a in row_args] + [const_spec(c) for c in const_args]
    out_specs = tuple(pl.BlockSpec((block,) + s[1:],
                                   lambda i, nd=len(s): (i,) + (0,) * (nd - 1))
                      for s in out_shapes)
    out_shape = tuple(jax.ShapeDtypeStruct(s, jnp.float32) for s in out_shapes)
    n_in = len(row_args) + len(const_args)

    def body(*refs):
        ins = [r[...] for r in refs[:n_in]]
        outs = fn(*ins)
        if not isinstance(outs, (tuple, list)):
            outs = (outs,)
        for oref, val in zip(refs[n_in:], outs):
            oref[...] = val

    res = pl.pallas_call(
        body, grid=grid, in_specs=in_specs, out_specs=out_specs,
        out_shape=out_shape)(*row_args, *const_args)
    return res


def _row_reduce_sum(fn, row_args, const_args=(), block=2048):
    R = row_args[0].shape[0]
    grid = (R // block,)

    def row_spec(a):
        nd = a.ndim
        return pl.BlockSpec((block,) + a.shape[1:],
                            lambda i, nd=nd: (i,) + (0,) * (nd - 1))

    def const_spec(a):
        nd = a.ndim
        return pl.BlockSpec(a.shape, lambda i, nd=nd: (0,) * nd)

    in_specs = [row_spec(a) for a in row_args] + [const_spec(c) for c in const_args]
    n_in = len(row_args) + len(const_args)

    def body(*refs):
        i = pl.program_id(0)
        out = refs[-1]
        @pl.when(i == 0)
        def _():
            out[...] = jnp.zeros_like(out)
        ins = [r[...] for r in refs[:n_in]]
        out[...] += jnp.sum(fn(*ins)).reshape(1, 1)

    return pl.pallas_call(
        body, grid=grid, in_specs=in_specs,
        out_specs=pl.BlockSpec((1, 1), lambda i: (0, 0)),
        out_shape=jax.ShapeDtypeStruct((1, 1), jnp.float32))(*row_args, *const_args)


def _qmul(a, b):
    aw, ax, ay, az = a[:, 0], a[:, 1], a[:, 2], a[:, 3]
    bw, bx, by, bz = b[:, 0], b[:, 1], b[:, 2], b[:, 3]
    return jnp.stack([
        aw * bw - ax * bx - ay * by - az * bz,
        aw * bx + ax * bw + ay * bz - az * by,
        aw * by - ax * bz + ay * bw + az * bx,
        aw * bz + ax * by - ay * bx + az * bw], axis=1)


def _qmul_inv_a(a, b):
    ai = jnp.stack([a[:, 0], -a[:, 1], -a[:, 2], -a[:, 3]], axis=1)
    return _qmul(ai, b)


def _l2n(v):
    return v / (jnp.sqrt(jnp.sum(v * v, axis=1, keepdims=True)) + 1e-8)


def _node_mm(x, W1, W2):
    W12 = jnp.concatenate([W1, W2], axis=1)
    return _row_call(
        lambda xb, w: jnp.dot(xb, w, preferred_element_type=jnp.float32),
        [(x.shape[0], W12.shape[1])], [x], [W12])[0]


def _node_finish(aggp, cnts, F):
    NT = aggp.shape[0] // 2
    ccol = cnts.shape[1] - 16
    return _row_call(
        lambda a0, a1, c0, c1: jax.nn.relu(
            (a0 + a1)[:, :F] / jnp.maximum((c0 + c1)[:, ccol:ccol + 1], 1.0)),
        [(NT, F)], [aggp[:NT], aggp[NT:], cnts[:NT], cnts[NT:]])[0]


def _node_finish_cat(aggA, aggB, cnts, F):
    NT = aggA.shape[0] // 2
    ccol = cnts.shape[1] - 16
    H = F // 2

    def fn(a0, a1, b0, b1, c0, c1):
        cnt = jnp.maximum((c0 + c1)[:, ccol:ccol + 1], 1.0)
        return jax.nn.relu(
            jnp.concatenate([(a0 + a1)[:, :H], (b0 + b1)[:, :H]], axis=1) / cnt)
    return _row_call(fn, [(NT, F)],
                     [aggA[:NT], aggA[NT:], aggB[:NT], aggB[NT:],
                      cnts[:NT], cnts[NT:]])[0]


def _edge_p(ea_list, W3, b, mask=None, mask_mult=False, cnt_col=False):
    F = W3.shape[1]
    nea = len(ea_list)
    Fo = F + 16 if cnt_col else F

    def fn(*args):
        p = jnp.dot(jnp.concatenate(args[:nea], axis=1), args[-2],
                    preferred_element_type=jnp.float32) + args[-1]
        if mask_mult:
            p = p * args[nea]
        if cnt_col:
            z = jnp.zeros((p.shape[0], 15), jnp.float32)
            p = jnp.concatenate([p, args[nea], z], axis=1)
        return p
    rows = list(ea_list) + ([mask] if mask is not None else [])
    return _row_call(fn, [(ea_list[0].shape[0], Fo)], rows,
                     [W3, b.reshape(1, -1)])[0]


def _relu_rows(a, F):
    return _row_call(lambda v: jax.nn.relu(v[:, :F]), [(a.shape[0], F)], [a])[0]


CH = 128
NW = 32
NCH = EP // CH // NW


def _conv_scatter(tab, p, row2d, col2d, NT, F):
    Fp = p.shape[1]
    has_cnt = Fp == F + 16
    AW = Fp
    TW = tab.shape[1]
    rows_pt = NT // 16
    assert rows_pt % 32 == 0 and F % 16 == 0
    mesh = plsc.VectorSubcoreMesh(core_axis_name="c", subcore_axis_name="s")
    out_type = (jax.ShapeDtypeStruct((EP, F), jnp.float32),
                jax.ShapeDtypeStruct((2 * NT, AW), jnp.float32))
    scratch = [
        pltpu.VMEM((NCH, CH), jnp.int32),
        pltpu.VMEM((NCH, CH), jnp.int32),
        pltpu.VMEM((CH, TW), jnp.float32),
        pltpu.VMEM((CH, TW), jnp.float32),
        pltpu.VMEM((CH, Fp), jnp.float32),
        pltpu.VMEM((CH, AW), jnp.float32),
        pltpu.VMEM((32, AW), jnp.float32),
        pltpu.VMEM_SHARED((NT, AW), jnp.float32),
        pltpu.SemaphoreType.DMA,
        pltpu.SemaphoreType.DMA,
    ]
    if has_cnt:
        scratch.insert(6, pltpu.VMEM((CH, F), jnp.float32))

    def body(*refs):
        if has_cnt:
            (tab_h, p_h, r_h, c_h, e_h, agg_h,
             rbuf, cbuf, gr, gc, pb, sb, eb, zb, agg_sh, sem1, sem2) = refs
        else:
            (tab_h, p_h, r_h, c_h, e_h, agg_h,
             rbuf, cbuf, gr, gc, pb, sb, zb, agg_sh, sem1, sem2) = refs
            eb = sb
        cid = lax.axis_index("c")
        sid = lax.axis_index("s")
        w = sid * 2 + cid
        ebase = w * NCH

        def zrow32(r, _):
            for q in range(AW // 16):
                zb[r, pl.ds(q * 16, 16)] = jnp.zeros((16,), jnp.float32)
            return 0
        lax.fori_loop(0, 32, zrow32, 0)
        for k in range(rows_pt // 32):
            pltpu.sync_copy(zb, agg_sh.at[pl.ds(sid * rows_pt + k * 32, 32)])
        plsc.subcore_barrier()

        pltpu.sync_copy(r_h.at[pl.ds(ebase, NCH)], rbuf)
        pltpu.sync_copy(c_h.at[pl.ds(ebase, NCH)], cbuf)

        def chunk(j, _):
            base = (ebase + j) * CH
            cp1 = pltpu.async_copy(tab_h.at[rbuf.at[j]], gr, sem1)
            cp2 = pltpu.async_copy(tab_h.at[cbuf.at[j]], gc, sem2)
            pltpu.sync_copy(p_h.at[pl.ds(base, CH)], pb)
            cp1.wait()
            cp2.wait()

            def crow(r, _):
                for q in range(F // 16):
                    s = pl.ds(q * 16, 16)
                    v = gr[r, s] + gc[r, pl.ds(TW // 2 + q * 16, 16)] + pb[r, s]
                    sb[r, s] = v
                    if has_cnt:
                        eb[r, s] = v
                if has_cnt:
                    sb[r, pl.ds(F, 16)] = pb[r, pl.ds(F, 16)]
                return 0
            lax.fori_loop(0, CH, crow, 0)

            pltpu.sync_copy(eb, e_h.at[pl.ds(base, CH)])
            pltpu.sync_copy(sb, agg_sh.at[cbuf.at[j]], add=True)
            return 0
        lax.fori_loop(0, NCH, chunk, 0)
        plsc.subcore_barrier()

        for k in range(rows_pt // 32):
            r0 = sid * rows_pt + k * 32
            pltpu.sync_copy(agg_sh.at[pl.ds(r0, 32)], agg_h.at[pl.ds(cid * NT + r0, 32)])

    kfn = pl.kernel(body, out_type=out_type, mesh=mesh, scratch_types=scratch,
                    compiler_params=pltpu.CompilerParams(use_tc_tiling_on_sc=False))
    return kfn(tab, p, row2d, col2d)


def _gather_rows(tab, idx):
    return tab[idx]


def _scatter_rows_add(base, idx, delta):
    return base.at[idx].add(delta)


def _gat_scores(Wg, a_s, a_d, x, rowi, coli, m, n_out):
    h = (x @ Wg)[:, 0]
    e = jax.nn.leaky_relu(a_s * h[rowi] + a_d * h[coli], 0.2)
    e = jnp.where(m > 0, e, -1e9)
    mx = jnp.full((n_out,), -1e9, jnp.float32).at[coli].max(e)
    ex = jnp.exp(e - mx[coli]) * m
    den = jnp.zeros((n_out,), jnp.float32).at[coli].add(ex)
    alpha = ex / jnp.maximum(den[coli], 1e-9)
    gat = jnp.zeros((n_out,), jnp.float32).at[coli].add(alpha * h[rowi])
    return jnp.tanh(gat)


def _pool_remap(perm, k, rowi, coli, m, n_nodes):
    kept = jnp.zeros((n_nodes,), bool).at[perm].set(True)
    nid = jnp.zeros((n_nodes,), jnp.int32).at[perm].set(jnp.arange(k, dtype=jnp.int32))
    valid = kept[jnp.minimum(rowi, n_nodes - 1)] & kept[jnp.minimum(coli, n_nodes - 1)] & (m > 0)
    row2 = jnp.where(valid, nid[jnp.minimum(rowi, n_nodes - 1)], k)
    col2 = jnp.where(valid, nid[jnp.minimum(coli, n_nodes - 1)], k)
    fm = valid.astype(jnp.float32)
    return row2, col2, fm


def kernel(x_org, edge_index, edge_attr, gt_q, beta, params):
    p_ = params
    rowi = jnp.concatenate([edge_index[0], jnp.full((EP - E,), N, jnp.int32)])
    coli = jnp.concatenate([edge_index[1], jnp.full((EP - E,), N, jnp.int32)])
    ones_m = jnp.concatenate([jnp.ones((E,), jnp.float32), jnp.zeros((EP - E,), jnp.float32)])
    row2d, col2d = rowi.reshape(-1, CH), coli.reshape(-1, CH)
    ones2d = ones_m.reshape(-1, CH)
    ea8 = jnp.pad(edge_attr, ((0, EP - E), (0, 0)))
    x_org_p = jnp.pad(x_org, ((0, NP - N), (0, 0)))
    gt_q_p = jnp.pad(gt_q, ((0, NP - N), (0, 0)))
    k1, k2 = N // 2, N // 4
    K1P, K2P = 5120, 2560

    x0r = _gather_rows(x_org_p, rowi)
    x0c = _gather_rows(x_org_p, coli)
    eam = _row_call(
        lambda a, bq, e4: _l2n(_qmul(e4, _qmul_inv_a(a, bq))),
        [(EP, 4)], [x0r, x0c, ea8[:, :4]])[0]

    W, b = p_['c1W'], p_['c1b']
    ones_c = ones_m.reshape(-1, 1)
    tab = _node_mm(x_org_p, W[0:4], W[4:8])
    p1 = _edge_p([eam], W[8:12], b, mask=ones_c, cnt_col=True)
    e1p, agg1p = _conv_scatter(tab, p1, row2d, col2d, NP, 32)
    x1 = _node_finish(agg1p, agg1p, 32)
    e1r = _relu_rows(e1p, 32)

    W, b = p_['c2W'], p_['c2b']
    tab = _node_mm(x1, W[0:32], W[32:64])
    p2 = _edge_p([eam, e1r], W[64:100], b)
    e2p, aggp = _conv_scatter(tab, p2, row2d, col2d, NP, 32)
    x2 = _node_finish(aggp, agg1p, 32)
    e2r = _relu_rows(e2p, 32)

    W, b = p_['c3W'], p_['c3b']
    xc = jnp.concatenate([x2, x1], axis=1)
    tab = _node_mm(xc, W[0:64], W[64:128])
    p3 = _edge_p([e2r, e1r], W[128:192], b)
    e3p, aggp = _conv_scatter(tab, p3, row2d, col2d, NP, 32)
    x3 = _node_finish(aggp, agg1p, 32)
    e3r = _relu_rows(e3p, 32)

    W, b = p_['spW'], p_['spb']
    tabA = _node_mm(x3, W[0:32, :32], W[32:64, :32])
    tabB = _node_mm(x3, W[0:32, 32:], W[32:64, 32:])
    pspA = _edge_p([e3r], W[64:96, :32], b[:32])
    pspB = _edge_p([e3r], W[64:96, 32:], b[32:])
    espA, aggA = _conv_scatter(tabA, pspA, row2d, col2d, NP, 32)
    espB, aggB = _conv_scatter(tabB, pspB, row2d, col2d, NP, 32)
    xs1 = _node_finish_cat(aggA, aggB, agg1p, 64)
    es1rA = _relu_rows(espA, 32)
    es1rB = _relu_rows(espB, 32)

    score1 = _gat_scores(p_['g1Wg'], p_['g1as'], p_['g1ad'], xs1, rowi, coli, ones_m, NP)
    vals1, perm1 = lax.top_k(score1[:N], k1)
    row2, col2, fm1 = _pool_remap(perm1, k1, rowi, coli, ones_m, N)
    xs1g = _gather_rows(xs1, perm1)
    xs1p = jnp.pad(xs1g * vals1[:, None], ((0, K1P - k1), (0, 0)))

    W, b = p_['s1W'], p_['s1b']
    fm1c = fm1.reshape(-1, 1)
    row2d2, col2d2 = row2.reshape(-1, CH), col2.reshape(-1, CH)
    tabA = _node_mm(xs1p, W[0:64, :32], W[64:128, :32])
    tabB = _node_mm(xs1p, W[0:64, 32:], W[64:128, 32:])
    ps1A = _edge_p([es1rA, es1rB], W[128:192, :32], b[:32], mask=fm1c, mask_mult=True)
    ps1B = _edge_p([es1rA, es1rB], W[128:192, 32:], b[32:], mask=fm1c,
                   mask_mult=True, cnt_col=True)
    es2A, aggA = _conv_scatter(tabA, ps1A, row2d2, col2d2, K1P, 32)
    es2B, aggs1p = _conv_scatter(tabB, ps1B, row2d2, col2d2, K1P, 32)
    xs2 = _node_finish_cat(aggA, aggs1p, aggs1p, 64)
    es2rA = _relu_rows(es2A, 32)
    es2rB = _relu_rows(es2B, 32)

    score2 = _gat_scores(p_['g2Wg'], p_['g2as'], p_['g2ad'], xs2, row2, col2, fm1, K1P)
    vals2, perm2 = lax.top_k(score2[:k1], k2)
    row3, col3, fm2 = _pool_remap(perm2, k2, row2, col2, fm1, k1)
    xs2g = _gather_rows(xs2, perm2)
    xssp = jnp.pad(xs2g * vals2[:, None], ((0, K2P - k2), (0, 0)))

    W, b = p_['ssW'], p_['ssb']
    fm2c = fm2.reshape(-1, 1)
    row2d3, col2d3 = row3.reshape(-1, CH), col3.reshape(-1, CH)
    tab = _node_mm(xssp, W[0:64], W[64:128])
    pss1 = _edge_p([es2rA, es2rB], W[128:192], b, mask=fm2c, mask_mult=True, cnt_col=True)
    ess1_p, aggss1p = _conv_scatter(tab, pss1, row2d3, col2d3, K2P, 64)
    xss1 = _node_finish(aggss1p, aggss1p, 64)
    ess1r = _relu_rows(ess1_p, 64)

    tab = _node_mm(xss1, W[0:64], W[64:128])
    pss2 = _edge_p([ess1r], W[128:192], b, mask=fm2c, mask_mult=True)
    ess2_p, aggp = _conv_scatter(tab, pss2, row2d3, col2d3, K2P, 64)
    xss2 = _node_finish(aggp, aggss1p, 64)

    xs2m = _scatter_rows_add(xs2, perm2, xss2[:k2])

    W, b = p_['s2W'], p_['s2b']
    tab = _node_mm(xs2m, W[0:64], W[64:128])
    ps2 = _edge_p([es2rA, es2rB], W[128:192], b, mask=fm1c, mask_mult=True)
    es3_p, aggp = _conv_scatter(tab, ps2, row2d2, col2d2, K1P, 32)
    xs3 = _node_finish(aggp, aggs1p, 32)

    x3b = _scatter_rows_add(x3, perm1, xs3[:k1])

    W, b = p_['c4W'], p_['c4b']
    xc4 = jnp.concatenate([x3b, x2], axis=1)
    tab = _node_mm(xc4, W[0:64], W[64:128])
    p4 = _edge_p([e3r, e2r], W[128:192], b)
    e4p, aggp = _conv_scatter(tab, p4, row2d, col2d, NP, 32)
    x4 = _node_finish(aggp, agg1p, 32)
    e4r = _relu_rows(e4p, 32)

    xc5 = jnp.concatenate([x4, x3b], axis=1)
    tab = _node_mm(xc5, W[0:64], W[64:128])
    p5 = _edge_p([e4r, e3r], W[128:192], b)
    e5p, aggp = _conv_scatter(tab, p5, row2d, col2d, NP, 32)
    x5 = _node_finish(aggp, agg1p, 32)
    e5r = _relu_rows(e5p, 32)

    xq = _row_call(
        lambda xv, xo, w, bb: _l2n(_qmul(jnp.dot(xv, w, preferred_element_type=jnp.float32) + bb, xo)),
        [(NP, 4)], [x5, x_org_p], [p_['l1W'], p_['l1b'].reshape(1, -1)])[0]

    gqr = _gather_rows(gt_q_p, rowi)
    gqc = _gather_rows(gt_q_p, coli)
    xr = _gather_rows(xq, rowi)
    xc_ = _gather_rows(xq, coli)

    def loss_fn(gr, gc, ar, ac, mv, bt):
        qa = _qmul_inv_a(gr, gc)
        qb = _qmul_inv_a(ar, ac)
        lv = _l2n(_qmul_inv_a(qa, qb))
        d = jnp.abs(jnp.stack([lv[:, 0] - 1.0, lv[:, 1], lv[:, 2], lv[:, 3]], axis=1))
        bb = jnp.maximum(bt[0, 0], 1e-6)
        contrib = jnp.where(d < bb, 0.5 * d * d / bb, d - 0.5 * bb)
        return contrib * mv

    loss_sum = _row_reduce_sum(loss_fn, [gqr, gqc, xr, xc_, ones_m.reshape(-1, 1)],
                               [beta.reshape(1, 1)])
    loss1 = loss_sum[0, 0] / (E * 4.0)

    return (xq[:N], loss1, beta,
            (x1[:N], x2[:N], x3b[:N], x4[:N], x5[:N]),
            (e1r[:E], e2r[:E], e3r[:E], e4r[:E], e5r[:E]))


# speedup vs baseline: 1.1854x; 1.1165x over previous
import functools

import jax
import jax.numpy as jnp
from jax import lax
from jax.experimental import pallas as pl
from jax.experimental.pallas import tpu as pltpu
from jax.experimental.pallas import tpu_sc as plsc

N = 10000
E = 160000
NP = 10240
EP = 163840


def _row_call(fn, out_shapes, row_args, const_args=(), block=2048):
    R = row_args[0].shape[0]
    while R % block:
        block //= 2
    grid = (R // block,)

    def row_spec(a):
        nd = a.ndim
        return pl.BlockSpec((block,) + a.shape[1:],
                            lambda i, nd=nd: (i,) + (0,) * (nd - 1))

    def const_spec(a):
        nd = a.ndim
        return pl.BlockSpec(a.shape, lambda i, nd=nd: (0,) * nd)

    in_specs = [row_spec(a) for a in row_args] + [const_spec(c) for c in const_args]
    out_specs = tuple(pl.BlockSpec((block,) + s[1:],
                                   lambda i, nd=len(s): (i,) + (0,) * (nd - 1))
                      for s in out_shapes)
    out_shape = tuple(jax.ShapeDtypeStruct(s, jnp.float32) for s in out_shapes)
    n_in = len(row_args) + len(const_args)

    def body(*refs):
        ins = [r[...] for r in refs[:n_in]]
        outs = fn(*ins)
        if not isinstance(outs, (tuple, list)):
            outs = (outs,)
        for oref, val in zip(refs[n_in:], outs):
            oref[...] = val

    res = pl.pallas_call(
        body, grid=grid, in_specs=in_specs, out_specs=out_specs,
        out_shape=out_shape)(*row_args, *const_args)
    return res


def _row_reduce_sum(fn, row_args, const_args=(), block=2048):
    R = row_args[0].shape[0]
    grid = (R // block,)

    def row_spec(a):
        nd = a.ndim
        return pl.BlockSpec((block,) + a.shape[1:],
                            lambda i, nd=nd: (i,) + (0,) * (nd - 1))

    def const_spec(a):
        nd = a.ndim
        return pl.BlockSpec(a.shape, lambda i, nd=nd: (0,) * nd)

    in_specs = [row_spec(a) for a in row_args] + [const_spec(c) for c in const_args]
    n_in = len(row_args) + len(const_args)

    def body(*refs):
        i = pl.program_id(0)
        out = refs[-1]
        @pl.when(i == 0)
        def _():
            out[...] = jnp.zeros_like(out)
        ins = [r[...] for r in refs[:n_in]]
        out[...] += jnp.sum(fn(*ins)).reshape(1, 1)

    return pl.pallas_call(
        body, grid=grid, in_specs=in_specs,
        out_specs=pl.BlockSpec((1, 1), lambda i: (0, 0)),
        out_shape=jax.ShapeDtypeStruct((1, 1), jnp.float32))(*row_args, *const_args)


def _qmul(a, b):
    aw, ax, ay, az = a[:, 0], a[:, 1], a[:, 2], a[:, 3]
    bw, bx, by, bz = b[:, 0], b[:, 1], b[:, 2], b[:, 3]
    return jnp.stack([
        aw * bw - ax * bx - ay * by - az * bz,
        aw * bx + ax * bw + ay * bz - az * by,
        aw * by - ax * bz + ay * bw + az * bx,
        aw * bz + ax * by - ay * bx + az * bw], axis=1)


def _qmul_inv_a(a, b):
    ai = jnp.stack([a[:, 0], -a[:, 1], -a[:, 2], -a[:, 3]], axis=1)
    return _qmul(ai, b)


def _l2n(v):
    return v / (jnp.sqrt(jnp.sum(v * v, axis=1, keepdims=True)) + 1e-8)


def _node_mm(x, W1, W2):
    W12 = jnp.concatenate([W1, W2], axis=1)
    return _row_call(
        lambda xb, w: jnp.dot(xb, w, preferred_element_type=jnp.float32),
        [(x.shape[0], W12.shape[1])], [x], [W12])[0]


def _node_finish(aggp, cnts, F):
    NT = aggp.shape[0] // 2
    ccol = cnts.shape[1] - 16
    return _row_call(
        lambda a0, a1, c0, c1: jax.nn.relu(
            (a0 + a1)[:, :F] / jnp.maximum((c0 + c1)[:, ccol:ccol + 1], 1.0)),
        [(NT, F)], [aggp[:NT], aggp[NT:], cnts[:NT], cnts[NT:]])[0]


def _node_finish_cat(aggA, aggB, cnts, F):
    NT = aggA.shape[0] // 2
    ccol = cnts.shape[1] - 16
    H = F // 2

    def fn(a0, a1, b0, b1, c0, c1):
        cnt = jnp.maximum((c0 + c1)[:, ccol:ccol + 1], 1.0)
        return jax.nn.relu(
            jnp.concatenate([(a0 + a1)[:, :H], (b0 + b1)[:, :H]], axis=1) / cnt)
    return _row_call(fn, [(NT, F)],
                     [aggA[:NT], aggA[NT:], aggB[:NT], aggB[NT:],
                      cnts[:NT], cnts[NT:]])[0]


def _edge_p(ea_list, W3, b, mask=None, mask_mult=False, cnt_col=False):
    F = W3.shape[1]
    nea = len(ea_list)
    Fo = F + 16 if cnt_col else F

    def fn(*args):
        p = jnp.dot(jnp.concatenate(args[:nea], axis=1), args[-2],
                    preferred_element_type=jnp.float32) + args[-1]
        if mask_mult:
            p = p * args[nea]
        if cnt_col:
            z = jnp.zeros((p.shape[0], 15), jnp.float32)
            p = jnp.concatenate([p, args[nea], z], axis=1)
        return p
    rows = list(ea_list) + ([mask] if mask is not None else [])
    return _row_call(fn, [(ea_list[0].shape[0], Fo)], rows,
                     [W3, b.reshape(1, -1)])[0]


def _relu_rows(a, F):
    return _row_call(lambda v: jax.nn.relu(v[:, :F]), [(a.shape[0], F)], [a])[0]


def _pack_cols(specs, R, W=32):
    arrs = [a for _, a in specs]

    def fn(*args):
        pieces = []
        pos = 0
        for (c, _), a in zip(specs, args):
            if c > pos:
                pieces.append(jnp.zeros((a.shape[0], c - pos), jnp.float32))
            pieces.append(a)
            pos = c + a.shape[1]
        if pos < W:
            pieces.append(jnp.zeros((args[0].shape[0], W - pos), jnp.float32))
        return jnp.concatenate(pieces, axis=1)
    return _row_call(fn, [(R, W)], arrs)[0]


CH = 128
NW = 32
NCH = EP // CH // NW


def _conv_scatter(tab, p, row2d, col2d, NT, F):
    Fp = p.shape[1]
    has_cnt = Fp == F + 16
    AW = Fp
    TW = tab.shape[1]
    rows_pt = NT // 16
    assert rows_pt % 32 == 0 and F % 16 == 0
    mesh = plsc.VectorSubcoreMesh(core_axis_name="c", subcore_axis_name="s")
    out_type = (jax.ShapeDtypeStruct((EP, F), jnp.float32),
                jax.ShapeDtypeStruct((2 * NT, AW), jnp.float32))
    scratch = [
        pltpu.VMEM((NCH, CH), jnp.int32),
        pltpu.VMEM((NCH, CH), jnp.int32),
        pltpu.VMEM((CH, TW), jnp.float32),
        pltpu.VMEM((CH, TW), jnp.float32),
        pltpu.VMEM((CH, Fp), jnp.float32),
        pltpu.VMEM((CH, AW), jnp.float32),
        pltpu.VMEM((32, AW), jnp.float32),
        pltpu.VMEM_SHARED((NT, AW), jnp.float32),
        pltpu.SemaphoreType.DMA,
        pltpu.SemaphoreType.DMA,
    ]
    if has_cnt:
        scratch.insert(6, pltpu.VMEM((CH, F), jnp.float32))

    def body(*refs):
        if has_cnt:
            (tab_h, p_h, r_h, c_h, e_h, agg_h,
             rbuf, cbuf, gr, gc, pb, sb, eb, zb, agg_sh, sem1, sem2) = refs
        else:
            (tab_h, p_h, r_h, c_h, e_h, agg_h,
             rbuf, cbuf, gr, gc, pb, sb, zb, agg_sh, sem1, sem2) = refs
            eb = sb
        cid = lax.axis_index("c")
        sid = lax.axis_index("s")
        w = sid * 2 + cid
        ebase = w * NCH

        def zrow32(r, _):
            for q in range(AW // 16):
                zb[r, pl.ds(q * 16, 16)] = jnp.zeros((16,), jnp.float32)
            return 0
        lax.fori_loop(0, 32, zrow32, 0)
        for k in range(rows_pt // 32):
            pltpu.sync_copy(zb, agg_sh.at[pl.ds(sid * rows_pt + k * 32, 32)])
        plsc.subcore_barrier()

        pltpu.sync_copy(r_h.at[pl.ds(ebase, NCH)], rbuf)
        pltpu.sync_copy(c_h.at[pl.ds(ebase, NCH)], cbuf)

        def chunk(j, _):
            base = (ebase + j) * CH
            cp1 = pltpu.async_copy(tab_h.at[rbuf.at[j]], gr, sem1)
            cp2 = pltpu.async_copy(tab_h.at[cbuf.at[j]], gc, sem2)
            pltpu.sync_copy(p_h.at[pl.ds(base, CH)], pb)
            cp1.wait()
            cp2.wait()

            def crow(r, _):
                for q in range(F // 16):
                    s = pl.ds(q * 16, 16)
                    v = gr[r, s] + gc[r, pl.ds(TW // 2 + q * 16, 16)] + pb[r, s]
                    sb[r, s] = v
                    if has_cnt:
                        eb[r, s] = v
                if has_cnt:
                    sb[r, pl.ds(F, 16)] = pb[r, pl.ds(F, 16)]
                return 0
            lax.fori_loop(0, CH, crow, 0)

            pltpu.sync_copy(eb, e_h.at[pl.ds(base, CH)])
            pltpu.sync_copy(sb, agg_sh.at[cbuf.at[j]], add=True)
            return 0
        lax.fori_loop(0, NCH, chunk, 0)
        plsc.subcore_barrier()

        for k in range(rows_pt // 32):
            r0 = sid * rows_pt + k * 32
            pltpu.sync_copy(agg_sh.at[pl.ds(r0, 32)], agg_h.at[pl.ds(cid * NT + r0, 32)])

    kfn = pl.kernel(body, out_type=out_type, mesh=mesh, scratch_types=scratch,
                    compiler_params=pltpu.CompilerParams(use_tc_tiling_on_sc=False))
    return kfn(tab, p, row2d, col2d)


def _gather_rows(tab, idx):
    return tab[idx]


def _scatter_rows_add(base, idx, delta):
    return base.at[idx].add(delta)


def _gat_scores(Wg, a_s, a_d, x, rowi, coli, m, n_out):
    h = (x @ Wg)[:, 0]
    e = jax.nn.leaky_relu(a_s * h[rowi] + a_d * h[coli], 0.2)
    ex = jnp.exp(e) * m
    den = jnp.zeros((n_out,), jnp.float32).at[coli].add(ex)
    num = jnp.zeros((n_out,), jnp.float32).at[coli].add(ex * h[rowi])
    return jnp.tanh(num / jnp.maximum(den, 1e-9))


def _pool_remap(perm, k, rowi, coli, m, n_nodes):
    kept = jnp.zeros((n_nodes,), bool).at[perm].set(True)
    nid = jnp.zeros((n_nodes,), jnp.int32).at[perm].set(jnp.arange(k, dtype=jnp.int32))
    valid = kept[jnp.minimum(rowi, n_nodes - 1)] & kept[jnp.minimum(coli, n_nodes - 1)] & (m > 0)
    row2 = jnp.where(valid, nid[jnp.minimum(rowi, n_nodes - 1)], k)
    col2 = jnp.where(valid, nid[jnp.minimum(coli, n_nodes - 1)], k)
    fm = valid.astype(jnp.float32)
    return row2.reshape(-1, CH), col2.reshape(-1, CH), fm


def kernel(x_org, edge_index, edge_attr, gt_q, beta, params):
    p_ = params
    rowi = jnp.concatenate([edge_index[0], jnp.full((EP - E,), N, jnp.int32)])
    coli = jnp.concatenate([edge_index[1], jnp.full((EP - E,), N, jnp.int32)])
    ones_m = jnp.concatenate([jnp.ones((E,), jnp.float32), jnp.zeros((EP - E,), jnp.float32)])
    row2d, col2d = rowi.reshape(-1, CH), coli.reshape(-1, CH)
    ones2d = ones_m.reshape(-1, CH)
    ea8 = jnp.pad(edge_attr, ((0, EP - E), (0, 0)))
    x_org_p = jnp.pad(x_org, ((0, NP - N), (0, 0)))
    gt_q_p = jnp.pad(gt_q, ((0, NP - N), (0, 0)))
    k1, k2 = N // 2, N // 4
    K1P, K2P = 5120, 2560
    zp16 = jnp.zeros((EP, 16), jnp.float32)
    ztabNP = jnp.zeros((NP, 32), jnp.float32)
    ztabK1P = jnp.zeros((K1P, 32), jnp.float32)

    xtab = _pack_cols([(0, x_org_p), (20, x_org_p)], NP)
    exy, _ = _conv_scatter(xtab, zp16, row2d, col2d, NP, 16)
    eam = _row_call(
        lambda ev, e4: _l2n(_qmul(e4, _qmul_inv_a(ev[:, 0:4], ev[:, 4:8]))),
        [(EP, 4)], [exy, ea8[:, :4]])[0]

    W, b = p_['c1W'], p_['c1b']
    ones_c = ones_m.reshape(-1, 1)
    tab = _node_mm(x_org_p, W[0:4], W[4:8])
    p1 = _edge_p([eam], W[8:12], b, mask=ones_c, cnt_col=True)
    e1p, agg1p = _conv_scatter(tab, p1, row2d, col2d, NP, 32)
    x1 = _node_finish(agg1p, agg1p, 32)
    e1r = _relu_rows(e1p, 32)

    W, b = p_['c2W'], p_['c2b']
    tab = _node_mm(x1, W[0:32], W[32:64])
    p2 = _edge_p([eam, e1r], W[64:100], b)
    e2p, aggp = _conv_scatter(tab, p2, row2d, col2d, NP, 32)
    x2 = _node_finish(aggp, agg1p, 32)
    e2r = _relu_rows(e2p, 32)

    W, b = p_['c3W'], p_['c3b']
    xc = jnp.concatenate([x2, x1], axis=1)
    tab = _node_mm(xc, W[0:64], W[64:128])
    p3 = _edge_p([e2r, e1r], W[128:192], b)
    e3p, aggp = _conv_scatter(tab, p3, row2d, col2d, NP, 32)
    x3 = _node_finish(aggp, agg1p, 32)
    e3r = _relu_rows(e3p, 32)

    W, b = p_['spW'], p_['spb']
    tabA = _node_mm(x3, W[0:32, :32], W[32:64, :32])
    tabB = _node_mm(x3, W[0:32, 32:], W[32:64, 32:])
    pspA = _edge_p([e3r], W[64:96, :32], b[:32])
    pspB = _edge_p([e3r], W[64:96, 32:], b[32:])
    espA, aggA = _conv_scatter(tabA, pspA, row2d, col2d, NP, 32)
    espB, aggB = _conv_scatter(tabB, pspB, row2d, col2d, NP, 32)
    xs1 = _node_finish_cat(aggA, aggB, agg1p, 64)
    es1rA = _relu_rows(espA, 32)
    es1rB = _relu_rows(espB, 32)

    score1 = _gat_scores(p_['g1Wg'], p_['g1as'], p_['g1ad'], xs1,
                         rowi, coli, ones_m, NP)
    vals1, perm1 = lax.top_k(score1[:N], k1)
    row2d2, col2d2, fm1 = _pool_remap(perm1, k1, rowi, coli, ones_m, NP)
    xs1g = _gather_rows(xs1, perm1)
    xs1p = jnp.pad(xs1g * vals1[:, None], ((0, K1P - k1), (0, 0)))

    W, b = p_['s1W'], p_['s1b']
    fm1c = fm1.reshape(-1, 1)
    tabA = _node_mm(xs1p, W[0:64, :32], W[64:128, :32])
    tabB = _node_mm(xs1p, W[0:64, 32:], W[64:128, 32:])
    ps1A = _edge_p([es1rA, es1rB], W[128:192, :32], b[:32], mask=fm1c, mask_mult=True)
    ps1B = _edge_p([es1rA, es1rB], W[128:192, 32:], b[32:], mask=fm1c,
                   mask_mult=True, cnt_col=True)
    es2A, aggA = _conv_scatter(tabA, ps1A, row2d2, col2d2, K1P, 32)
    es2B, aggs1p = _conv_scatter(tabB, ps1B, row2d2, col2d2, K1P, 32)
    xs2 = _node_finish_cat(aggA, aggs1p, aggs1p, 64)
    es2rA = _relu_rows(es2A, 32)
    es2rB = _relu_rows(es2B, 32)

    row2f, col2f = row2d2.reshape(-1), col2d2.reshape(-1)
    score2 = _gat_scores(p_['g2Wg'], p_['g2as'], p_['g2ad'], xs2,
                         row2f, col2f, fm1, K1P)
    vals2, perm2 = lax.top_k(score2[:k1], k2)
    row2d3, col2d3, fm2 = _pool_remap(perm2, k2, row2f, col2f, fm1, K1P)
    xs2g = _gather_rows(xs2, perm2)
    xssp = jnp.pad(xs2g * vals2[:, None], ((0, K2P - k2), (0, 0)))

    W, b = p_['ssW'], p_['ssb']
    fm2c = fm2.reshape(-1, 1)
    tab = _node_mm(xssp, W[0:64], W[64:128])
    pss1 = _edge_p([es2rA, es2rB], W[128:192], b, mask=fm2c, mask_mult=True, cnt_col=True)
    ess1_p, aggss1p = _conv_scatter(tab, pss1, row2d3, col2d3, K2P, 64)
    xss1 = _node_finish(aggss1p, aggss1p, 64)
    ess1r = _relu_rows(ess1_p, 64)

    tab = _node_mm(xss1, W[0:64], W[64:128])
    pss2 = _edge_p([ess1r], W[128:192], b, mask=fm2c, mask_mult=True)
    ess2_p, aggp = _conv_scatter(tab, pss2, row2d3, col2d3, K2P, 64)
    xss2 = _node_finish(aggp, aggss1p, 64)

    xs2m = _scatter_rows_add(xs2, perm2, xss2[:k2])

    W, b = p_['s2W'], p_['s2b']
    tab = _node_mm(xs2m, W[0:64], W[64:128])
    ps2 = _edge_p([es2rA, es2rB], W[128:192], b, mask=fm1c, mask_mult=True)
    es3_p, aggp = _conv_scatter(tab, ps2, row2d2, col2d2, K1P, 32)
    xs3 = _node_finish(aggp, aggs1p, 32)

    x3b = _scatter_rows_add(x3, perm1, xs3[:k1])

    W, b = p_['c4W'], p_['c4b']
    xc4 = jnp.concatenate([x3b, x2], axis=1)
    tab = _node_mm(xc4, W[0:64], W[64:128])
    p4 = _edge_p([e3r, e2r], W[128:192], b)
    e4p, aggp = _conv_scatter(tab, p4, row2d, col2d, NP, 32)
    x4 = _node_finish(aggp, agg1p, 32)
    e4r = _relu_rows(e4p, 32)

    xc5 = jnp.concatenate([x4, x3b], axis=1)
    tab = _node_mm(xc5, W[0:64], W[64:128])
    p5 = _edge_p([e4r, e3r], W[128:192], b)
    e5p, aggp = _conv_scatter(tab, p5, row2d, col2d, NP, 32)
    x5 = _node_finish(aggp, agg1p, 32)
    e5r = _relu_rows(e5p, 32)

    xq = _row_call(
        lambda xv, xo, w, bb: _l2n(_qmul(jnp.dot(xv, w, preferred_element_type=jnp.float32) + bb, xo)),
        [(NP, 4)], [x5, x_org_p], [p_['l1W'], p_['l1b'].reshape(1, -1)])[0]

    ltab = _pack_cols([(0, gt_q_p), (8, xq), (20, gt_q_p), (28, xq)], NP)
    lv16, _ = _conv_scatter(ltab, zp16, row2d, col2d, NP, 16)

    def loss_fn(ev, mv, bt):
        qa = _qmul_inv_a(ev[:, 0:4], ev[:, 4:8])
        qb = _qmul_inv_a(ev[:, 8:12], ev[:, 12:16])
        lv = _l2n(_qmul_inv_a(qa, qb))
        d = jnp.abs(jnp.stack([lv[:, 0] - 1.0, lv[:, 1], lv[:, 2], lv[:, 3]], axis=1))
        bb = jnp.maximum(bt[0, 0], 1e-6)
        contrib = jnp.where(d < bb, 0.5 * d * d / bb, d - 0.5 * bb)
        return contrib * mv

    loss_sum = _row_reduce_sum(loss_fn, [lv16, ones_m.reshape(-1, 1)],
                               [beta.reshape(1, 1)])
    loss1 = loss_sum[0, 0] / (E * 4.0)

    return (xq[:N], loss1, beta,
            (x1[:N], x2[:N], x3b[:N], x4[:N], x5[:N]),
            (e1r[:E], e2r[:E], e3r[:E], e4r[:E], e5r[:E]))
